# baseline (speedup 1.0000x reference)
import functools

import jax
import jax.numpy as jnp
from jax import lax
from jax.experimental import pallas as pl
from jax.experimental.pallas import tpu as pltpu
from jax.experimental.pallas import tpu_sc as plsc

_TOTAL = 64 * 64 * 4096
_NW = 32
_PER_W = _TOTAL // _NW
_CHUNK = 32768
_NCH = _PER_W // _CHUNK


def _sc_copy(x_hbm, out_hbm, buf0, buf1, sin, sout):
    wid = lax.axis_index("s") * 2 + lax.axis_index("c")
    base = wid * _PER_W
    bufs = (buf0, buf1)

    def cp_in(i):
        return pltpu.make_async_copy(
            x_hbm.at[pl.ds(base + i * _CHUNK, _CHUNK)], bufs[i % 2], sin)

    def cp_out(i):
        return pltpu.make_async_copy(
            bufs[i % 2], out_hbm.at[pl.ds(base + i * _CHUNK, _CHUNK)], sout)

    cp_in(0).start()
    for i in range(_NCH):
        cp_in(i).wait()
        cp_out(i).start()
        if i + 1 < _NCH:
            if i - 1 >= 0:
                cp_out(i - 1).wait()
            cp_in(i + 1).start()
    cp_out(_NCH - 1).wait()


def kernel(x):
    mesh = plsc.VectorSubcoreMesh(core_axis_name="c", subcore_axis_name="s")
    k = functools.partial(
        pl.kernel,
        out_type=jax.ShapeDtypeStruct((_TOTAL,), jnp.float32),
        mesh=mesh,
        scratch_types=[
            pltpu.VMEM((_CHUNK,), jnp.float32),
            pltpu.VMEM((_CHUNK,), jnp.float32),
            pltpu.SemaphoreType.DMA,
            pltpu.SemaphoreType.DMA,
        ],
    )(_sc_copy)
    return k(x.reshape(-1)).reshape(x.shape)


# speedup vs baseline: 4.4500x; 4.4500x over previous
import jax
import jax.numpy as jnp
from jax.experimental import pallas as pl

_GRID = 8


def _copy_block(x_ref, o_ref):
    o_ref[...] = x_ref[...]


def kernel(x):
    b, c, f = x.shape
    blk = b // _GRID
    return pl.pallas_call(
        _copy_block,
        grid=(_GRID,),
        in_specs=[pl.BlockSpec((blk, c, f), lambda i: (i, 0, 0))],
        out_specs=pl.BlockSpec((blk, c, f), lambda i: (i, 0, 0)),
        out_shape=jax.ShapeDtypeStruct((b, c, f), x.dtype),
    )(x)


# speedup vs baseline: 4.4629x; 1.0029x over previous
import jax
import jax.numpy as jnp
from jax.experimental import pallas as pl
from jax.experimental.pallas import tpu as pltpu

_N = 8
_B = 6
_D = 3


def _dma_pipeline(x_ref, o_ref, buf, sin, sout):
    rows = x_ref.shape[0] // _N

    def cp_in(i):
        return pltpu.make_async_copy(
            x_ref.at[pl.ds(i * rows, rows)], buf.at[i % _B], sin.at[i])

    def cp_out(i):
        return pltpu.make_async_copy(
            buf.at[i % _B], o_ref.at[pl.ds(i * rows, rows)], sout.at[i])

    for j in range(_D):
        cp_in(j).start()
    for i in range(_N):
        cp_in(i).wait()
        cp_out(i).start()
        j = i + _D
        if j < _N:
            if j - _B >= 0:
                cp_out(j - _B).wait()
            cp_in(j).start()
    for i in range(_N - _B, _N):
        cp_out(i).wait()


def kernel(x):
    b, c, f = x.shape
    return pl.pallas_call(
        _dma_pipeline,
        in_specs=[pl.BlockSpec(memory_space=pl.ANY)],
        out_specs=pl.BlockSpec(memory_space=pl.ANY),
        out_shape=jax.ShapeDtypeStruct((b, c, f), x.dtype),
        scratch_shapes=[
            pltpu.VMEM((_B, b // _N, c, f), x.dtype),
            pltpu.SemaphoreType.DMA((_N,)),
            pltpu.SemaphoreType.DMA((_N,)),
        ],
    )(x)


# speedup vs baseline: 4.4906x; 1.0062x over previous
import jax
import jax.numpy as jnp
from jax.experimental import pallas as pl
from jax.experimental.pallas import tpu as pltpu

_CHUNKS = (2, 2, 4, 8, 8, 8, 8, 8, 8, 4, 2, 2)
_N = len(_CHUNKS)
_B = 6
_D = 3
_OFFS = tuple(sum(_CHUNKS[:i]) for i in range(_N))


def _dma_pipeline(x_ref, o_ref, buf, sin, sout):
    def cp_in(i):
        return pltpu.make_async_copy(
            x_ref.at[pl.ds(_OFFS[i], _CHUNKS[i])],
            buf.at[i % _B, pl.ds(0, _CHUNKS[i])], sin.at[i])

    def cp_out(i):
        return pltpu.make_async_copy(
            buf.at[i % _B, pl.ds(0, _CHUNKS[i])],
            o_ref.at[pl.ds(_OFFS[i], _CHUNKS[i])], sout.at[i])

    for j in range(_D):
        cp_in(j).start()
    for i in range(_N):
        cp_in(i).wait()
        cp_out(i).start()
        j = i + _D
        if j < _N:
            if j - _B >= 0:
                cp_out(j - _B).wait()
            cp_in(j).start()
    for i in range(_N - _B, _N):
        cp_out(i).wait()


def kernel(x):
    b, c, f = x.shape
    return pl.pallas_call(
        _dma_pipeline,
        in_specs=[pl.BlockSpec(memory_space=pl.ANY)],
        out_specs=pl.BlockSpec(memory_space=pl.ANY),
        out_shape=jax.ShapeDtypeStruct((b, c, f), x.dtype),
        scratch_shapes=[
            pltpu.VMEM((_B, max(_CHUNKS), c, f), x.dtype),
            pltpu.SemaphoreType.DMA((_N,)),
            pltpu.SemaphoreType.DMA((_N,)),
        ],
    )(x)
